# Initial kernel scaffold; baseline (speedup 1.0000x reference)
import jax
import jax.numpy as jnp
from jax.experimental import pallas as pl


def kernel(x, vectors):
    raise NotImplementedError("write your pallas kernel here")


# speedup vs baseline: 4.0873x; 4.0873x over previous
import functools

import jax
import jax.numpy as jnp
from jax import lax
from jax.experimental import pallas as pl
from jax.experimental.pallas import tpu as pltpu
from jax.experimental.pallas import tpu_sc as plsc

VOCAB = 100000
EMBED_DIM = 64
BATCH = 4096
HIST = 50

_NUM_IDX = BATCH * HIST
_CHUNK = 128
_NUM_CHUNKS = _NUM_IDX // _CHUNK
_NW = 32
_CHUNKS_PER_W = _NUM_CHUNKS // _NW


@functools.partial(
    pl.kernel,
    mesh=plsc.VectorSubcoreMesh(core_axis_name="c", subcore_axis_name="s"),
    out_type=jax.ShapeDtypeStruct((_NUM_IDX, EMBED_DIM), jnp.float32),
    scratch_types=[
        pltpu.VMEM((_CHUNKS_PER_W, _CHUNK), jnp.int32),
        pltpu.VMEM((_CHUNK, EMBED_DIM), jnp.float32),
        pltpu.SemaphoreType.DMA,
    ],
    compiler_params=pltpu.CompilerParams(use_tc_tiling_on_sc=False),
)
def _gather_kernel(idx_hbm, table_hbm, out_hbm, idx_v, rows_v, sem):
    c = lax.axis_index("c")
    s = lax.axis_index("s")
    wid = s * 2 + c
    base_chunk = wid * _CHUNKS_PER_W
    pltpu.sync_copy(idx_hbm.at[wid], idx_v)

    def body(j, carry):
        pltpu.async_copy(table_hbm.at[idx_v.at[j]], rows_v, sem).wait()
        pltpu.sync_copy(
            rows_v, out_hbm.at[pl.ds((base_chunk + j) * _CHUNK, _CHUNK)]
        )
        return carry

    lax.fori_loop(0, _CHUNKS_PER_W, body, 0)


def kernel(x, vectors):
    idx = x.astype(jnp.int32).reshape(_NW, _CHUNKS_PER_W, _CHUNK)
    out = _gather_kernel(idx, vectors)
    return out.reshape(BATCH, HIST, EMBED_DIM)


# speedup vs baseline: 4.6761x; 1.1440x over previous
import functools

import jax
import jax.numpy as jnp
from jax import lax
from jax.experimental import pallas as pl
from jax.experimental.pallas import tpu as pltpu
from jax.experimental.pallas import tpu_sc as plsc

VOCAB = 100000
EMBED_DIM = 64
BATCH = 4096
HIST = 50

_NUM_IDX = BATCH * HIST
_CHUNK = 128
_NUM_CHUNKS = _NUM_IDX // _CHUNK
_NW = 32
_CHUNKS_PER_W = _NUM_CHUNKS // _NW
_NBUF = 5
_GROUPS = _CHUNKS_PER_W // _NBUF


@functools.partial(
    pl.kernel,
    mesh=plsc.VectorSubcoreMesh(core_axis_name="c", subcore_axis_name="s"),
    out_type=jax.ShapeDtypeStruct((_NUM_IDX, EMBED_DIM), jnp.float32),
    scratch_types=[
        pltpu.VMEM((_CHUNKS_PER_W, _CHUNK), jnp.int32),
        [pltpu.VMEM((_CHUNK, EMBED_DIM), jnp.float32) for _ in range(_NBUF)],
        [pltpu.SemaphoreType.DMA for _ in range(_NBUF)],
        [pltpu.SemaphoreType.DMA for _ in range(_NBUF)],
    ],
    compiler_params=pltpu.CompilerParams(use_tc_tiling_on_sc=False),
)
def _gather_kernel(idx_hbm, table_hbm, out_hbm, idx_v, bufs, g_sems, s_sems):
    c = lax.axis_index("c")
    s = lax.axis_index("s")
    wid = s * 2 + c
    base_chunk = wid * _CHUNKS_PER_W
    pltpu.sync_copy(idx_hbm.at[wid], idx_v)

    def g_copy(j, b):
        return pltpu.make_async_copy(table_hbm.at[idx_v.at[j]], bufs[b], g_sems[b])

    def s_copy(j, b):
        return pltpu.make_async_copy(
            bufs[b], out_hbm.at[pl.ds((base_chunk + j) * _CHUNK, _CHUNK)], s_sems[b]
        )

    for b in range(_NBUF):
        g_copy(b, b).start()

    def group(g, carry):
        for b in range(_NBUF):
            j = g * _NBUF + b
            g_copy(j, b).wait()
            s_copy(j, b).start()
            s_copy(j, b).wait()
            g_copy(j + _NBUF, b).start()
        return carry

    lax.fori_loop(0, _GROUPS - 1, group, 0)

    for b in range(_NBUF):
        j = (_GROUPS - 1) * _NBUF + b
        g_copy(j, b).wait()
        s_copy(j, b).start()
    for b in range(_NBUF):
        s_copy(b, b).wait()


def kernel(x, vectors):
    idx = x.astype(jnp.int32).reshape(_NW, _CHUNKS_PER_W, _CHUNK)
    out = _gather_kernel(idx, vectors)
    return out.reshape(BATCH, HIST, EMBED_DIM)
